# Initial kernel scaffold; baseline (speedup 1.0000x reference)
import jax
import jax.numpy as jnp
from jax.experimental import pallas as pl


def kernel(x, edge_index, edge_kpos, W_conv1, g0, b0, W1a, g1a, b1a, W1b, g1b, b1b, W2a, g2a, b2a, W2b, g2b, b2b, W2d, g2d, b2d, W3a, g3a, b3a, W3b, g3b, b3b, W3d, g3d, b3d, W_conv2):
    raise NotImplementedError("write your pallas kernel here")


# speedup vs baseline: 2.0543x; 2.0543x over previous
import functools

import jax
import jax.numpy as jnp
from jax import lax
from jax.experimental import pallas as pl
from jax.experimental.pallas import tpu as pltpu
from jax.experimental.pallas import tpu_sc as plsc

N = 10000
E = 320000
K = 27

_NC, _NS = 2, 16
_NW = _NC * _NS
_B = 128
_EW = 10112
_NCHUNK = _EW // _B
_E_PAD = _EW * _NW
_NACC = 10112
_RPT = _NACC // _NS
_PAD_DST = N


def _prep_indices(edge_index, edge_kpos):
    def body(ei_ref, kp_ref, g_ref, d_ref):
        src = ei_ref[0:1, :]
        dst = ei_ref[1:2, :]
        kp = kp_ref[...]
        g_ref[:, :E] = kp * N + src
        g_ref[:, E:] = jnp.zeros((1, _E_PAD - E), jnp.int32)
        d_ref[:, :E] = dst
        d_ref[:, E:] = jnp.full((1, _E_PAD - E), _PAD_DST, jnp.int32)

    gidx, dst = pl.pallas_call(
        body,
        out_shape=(
            jax.ShapeDtypeStruct((1, _E_PAD), jnp.int32),
            jax.ShapeDtypeStruct((1, _E_PAD), jnp.int32),
        ),
    )(edge_index, edge_kpos.reshape(1, E))
    return gidx.reshape(_E_PAD), dst.reshape(_E_PAD)


def _mm_all_k(h, W):
    C, O = W.shape[1], W.shape[2]

    def body(h_ref, w_ref, o_ref):
        o_ref[0] = jnp.dot(h_ref[...], w_ref[0],
                           preferred_element_type=jnp.float32)

    return pl.pallas_call(
        body,
        grid=(K,),
        in_specs=[
            pl.BlockSpec((N, C), lambda k: (0, 0)),
            pl.BlockSpec((1, C, O), lambda k: (k, 0, 0)),
        ],
        out_specs=pl.BlockSpec((1, N, O), lambda k: (k, 0, 0)),
        out_shape=jax.ShapeDtypeStruct((K, N, O), jnp.float32),
    )(h, W)


def _bn_cols(su, g, b):
    m = jnp.mean(su, axis=0, keepdims=True)
    v = jnp.mean((su - m) ** 2, axis=0, keepdims=True)
    return (su - m) * (g / jnp.sqrt(v + 1e-5)) + b


def _act(u, g, b, res=None, rbn=None, Wd=None):
    C = u.shape[-1]
    n_out = 1 if Wd is None else 2

    args = [u, g.reshape(1, C), b.reshape(1, C)]
    if res is not None:
        args.append(res)
    if rbn is not None:
        r, rg, rb = rbn
        args.extend([r, rg.reshape(1, -1), rb.reshape(1, -1)])
    out_shape = [jax.ShapeDtypeStruct((N, C), jnp.float32)]
    if Wd is not None:
        args.append(Wd)
        out_shape.append(jax.ShapeDtypeStruct((N, Wd.shape[1]), jnp.float32))

    def body2(*refs):
        nin = len(args)
        u_ref = refs[0]
        g_ref, b_ref = refs[1], refs[2]
        i = 3
        su = u_ref[0, :N, :] + u_ref[1, :N, :]
        h = _bn_cols(su, g_ref[...], b_ref[...])
        if res is not None:
            h = h + refs[i][...]
            i += 1
        if rbn is not None:
            h = h + _bn_cols(refs[i][...], refs[i + 1][...], refs[i + 2][...])
            i += 3
        h = jnp.maximum(h, 0.0)
        refs[nin][...] = h
        if Wd is not None:
            refs[nin + 1][...] = jnp.dot(h, refs[i][...],
                                         preferred_element_type=jnp.float32)

    outs = pl.pallas_call(
        body2,
        out_shape=tuple(out_shape) if n_out == 2 else out_shape[0],
    )(*args)
    return outs


def _final(u, g, b, r, rg, rb, W_conv2):
    C = u.shape[-1]

    def body(u_ref, g_ref, b_ref, r_ref, rg_ref, rb_ref, w_ref, o_ref):
        su = u_ref[0, :N, :] + u_ref[1, :N, :]
        h = _bn_cols(su, g_ref[...], b_ref[...])
        h = h + _bn_cols(r_ref[...], rg_ref[...], rb_ref[...])
        h = jnp.maximum(h, 0.0)
        o_ref[...] = jnp.dot(h, w_ref[...], preferred_element_type=jnp.float32)

    return pl.pallas_call(
        body,
        out_shape=jax.ShapeDtypeStruct((N, W_conv2.shape[1]), jnp.float32),
    )(u, g.reshape(1, C), b.reshape(1, C), r, rg.reshape(1, C),
      rb.reshape(1, C), W_conv2)


@functools.lru_cache(maxsize=None)
def _make_sc_scatter(O):
    mesh = plsc.VectorSubcoreMesh(core_axis_name="c", subcore_axis_name="s")

    @functools.partial(
        pl.kernel,
        mesh=mesh,
        compiler_params=pltpu.CompilerParams(use_tc_tiling_on_sc=False),
        out_type=jax.ShapeDtypeStruct((_NC, _NACC, O), jnp.float32),
        scratch_types=[
            pltpu.VMEM((_B,), jnp.int32),
            pltpu.VMEM((_B,), jnp.int32),
            pltpu.VMEM((_B, O), jnp.float32),
            pltpu.VMEM_SHARED((_NACC, O), jnp.float32),
            pltpu.SemaphoreType.DMA,
        ],
    )
    def sc_fn(xw_hbm, gidx_hbm, dst_hbm, zeros_hbm, out_hbm,
              idx_v, dst_v, rows_v, acc, sem):
        c = lax.axis_index("c")
        s = lax.axis_index("s")
        wid = c * _NS + s
        pltpu.sync_copy(zeros_hbm, acc.at[pl.ds(s * _RPT, _RPT)])
        plsc.subcore_barrier()

        def body(g, carry):
            base = pl.multiple_of(wid * _EW + g * _B, _B)
            pltpu.sync_copy(gidx_hbm.at[pl.ds(base, _B)], idx_v)
            pltpu.sync_copy(dst_hbm.at[pl.ds(base, _B)], dst_v)
            pltpu.async_copy(xw_hbm.at[idx_v], rows_v, sem).wait()
            pltpu.sync_copy(rows_v, acc.at[dst_v], add=True)
            return carry

        lax.fori_loop(0, _NCHUNK, body, 0)
        plsc.subcore_barrier()
        pltpu.sync_copy(acc.at[pl.ds(s * _RPT, _RPT)],
                        out_hbm.at[c, pl.ds(s * _RPT, _RPT)])

    return sc_fn


def _sconv_sc(xw, gidx, dst, zeros):
    O = xw.shape[-1]
    fn = _make_sc_scatter(O)
    return fn(xw.reshape(K * N, O), gidx, dst, zeros)


def kernel(x, edge_index, edge_kpos, W_conv1, g0, b0, W1a, g1a, b1a, W1b,
           g1b, b1b, W2a, g2a, b2a, W2b, g2b, b2b, W2d, g2d, b2d, W3a, g3a,
           b3a, W3b, g3b, b3b, W3d, g3d, b3d, W_conv2):
    gidx, dst = _prep_indices(edge_index, edge_kpos)
    z32 = jnp.zeros((_RPT, 32), jnp.float32)
    z64 = jnp.zeros((_RPT, 64), jnp.float32)
    z96 = jnp.zeros((_RPT, 96), jnp.float32)

    def sconv(h, W, zeros):
        return _sconv_sc(_mm_all_k(h, W), gidx, dst, zeros)

    u0 = sconv(x, W_conv1, z32)
    h0 = _act(u0, g0, b0)

    u1a = sconv(h0, W1a, z32)
    h1a = _act(u1a, g1a, b1a)
    u1b = sconv(h1a, W1b, z32)
    h1, r2 = _act(u1b, g1b, b1b, res=h0, Wd=W2d)

    u2a = sconv(h1, W2a, z64)
    h2a = _act(u2a, g2a, b2a)
    u2b = sconv(h2a, W2b, z64)
    h2, r3 = _act(u2b, g2b, b2b, rbn=(r2, g2d, b2d), Wd=W3d)

    u3a = sconv(h2, W3a, z96)
    h3a = _act(u3a, g3a, b3a)
    u3b = sconv(h3a, W3b, z96)

    return _final(u3b, g3b, b3b, r3, g3d, b3d, W_conv2)


# speedup vs baseline: 2.1111x; 1.0276x over previous
import functools

import jax
import jax.numpy as jnp
from jax import lax
from jax.experimental import pallas as pl
from jax.experimental.pallas import tpu as pltpu
from jax.experimental.pallas import tpu_sc as plsc

N = 10000
E = 320000
K = 27

_NC, _NS = 2, 16
_NW = _NC * _NS
_B = 128
_EW = 10240
_NCHUNK = _EW // _B
_E_PAD = _EW * _NW
_NBUF = 4
_NACC = 10112
_RPT = _NACC // _NS
_PAD_DST = N


def _prep_indices(edge_index, edge_kpos):
    def body(ei_ref, kp_ref, g_ref, d_ref):
        src = ei_ref[0:1, :]
        dst = ei_ref[1:2, :]
        kp = kp_ref[...]
        g_ref[:, :E] = kp * N + src
        g_ref[:, E:] = jnp.zeros((1, _E_PAD - E), jnp.int32)
        d_ref[:, :E] = dst
        d_ref[:, E:] = jnp.full((1, _E_PAD - E), _PAD_DST, jnp.int32)

    gidx, dst = pl.pallas_call(
        body,
        out_shape=(
            jax.ShapeDtypeStruct((1, _E_PAD), jnp.int32),
            jax.ShapeDtypeStruct((1, _E_PAD), jnp.int32),
        ),
    )(edge_index, edge_kpos.reshape(1, E))
    return gidx.reshape(_E_PAD), dst.reshape(_E_PAD)


def _mm_all_k(h, W):
    C, O = W.shape[1], W.shape[2]

    def body(h_ref, w_ref, o_ref):
        o_ref[0] = jnp.dot(h_ref[...], w_ref[0],
                           preferred_element_type=jnp.float32)

    return pl.pallas_call(
        body,
        grid=(K,),
        in_specs=[
            pl.BlockSpec((N, C), lambda k: (0, 0)),
            pl.BlockSpec((1, C, O), lambda k: (k, 0, 0)),
        ],
        out_specs=pl.BlockSpec((1, N, O), lambda k: (k, 0, 0)),
        out_shape=jax.ShapeDtypeStruct((K, N, O), jnp.float32),
    )(h, W)


def _bn_cols(su, g, b):
    m = jnp.mean(su, axis=0, keepdims=True)
    v = jnp.mean((su - m) ** 2, axis=0, keepdims=True)
    return (su - m) * (g / jnp.sqrt(v + 1e-5)) + b


def _act(u, g, b, res=None, rbn=None, Wd=None):
    C = u.shape[-1]
    n_out = 1 if Wd is None else 2

    args = [u, g.reshape(1, C), b.reshape(1, C)]
    if res is not None:
        args.append(res)
    if rbn is not None:
        r, rg, rb = rbn
        args.extend([r, rg.reshape(1, -1), rb.reshape(1, -1)])
    out_shape = [jax.ShapeDtypeStruct((N, C), jnp.float32)]
    if Wd is not None:
        args.append(Wd)
        out_shape.append(jax.ShapeDtypeStruct((N, Wd.shape[1]), jnp.float32))

    def body2(*refs):
        nin = len(args)
        u_ref = refs[0]
        g_ref, b_ref = refs[1], refs[2]
        i = 3
        su = u_ref[0, :N, :] + u_ref[1, :N, :]
        h = _bn_cols(su, g_ref[...], b_ref[...])
        if res is not None:
            h = h + refs[i][...]
            i += 1
        if rbn is not None:
            h = h + _bn_cols(refs[i][...], refs[i + 1][...], refs[i + 2][...])
            i += 3
        h = jnp.maximum(h, 0.0)
        refs[nin][...] = h
        if Wd is not None:
            refs[nin + 1][...] = jnp.dot(h, refs[i][...],
                                         preferred_element_type=jnp.float32)

    outs = pl.pallas_call(
        body2,
        out_shape=tuple(out_shape) if n_out == 2 else out_shape[0],
    )(*args)
    return outs


def _final(u, g, b, r, rg, rb, W_conv2):
    C = u.shape[-1]

    def body(u_ref, g_ref, b_ref, r_ref, rg_ref, rb_ref, w_ref, o_ref):
        su = u_ref[0, :N, :] + u_ref[1, :N, :]
        h = _bn_cols(su, g_ref[...], b_ref[...])
        h = h + _bn_cols(r_ref[...], rg_ref[...], rb_ref[...])
        h = jnp.maximum(h, 0.0)
        o_ref[...] = jnp.dot(h, w_ref[...], preferred_element_type=jnp.float32)

    return pl.pallas_call(
        body,
        out_shape=jax.ShapeDtypeStruct((N, W_conv2.shape[1]), jnp.float32),
    )(u, g.reshape(1, C), b.reshape(1, C), r, rg.reshape(1, C),
      rb.reshape(1, C), W_conv2)


@functools.lru_cache(maxsize=None)
def _make_sc_scatter(O):
    mesh = plsc.VectorSubcoreMesh(core_axis_name="c", subcore_axis_name="s")

    @functools.partial(
        pl.kernel,
        mesh=mesh,
        compiler_params=pltpu.CompilerParams(use_tc_tiling_on_sc=False),
        out_type=jax.ShapeDtypeStruct((_NC, _NACC, O), jnp.float32),
        scratch_types=[
            pltpu.VMEM((_NCHUNK, _B), jnp.int32),
            pltpu.VMEM((_NCHUNK, _B), jnp.int32),
            pltpu.VMEM((_NBUF, _B, O), jnp.float32),
            pltpu.VMEM_SHARED((_NACC, O), jnp.float32),
            pltpu.SemaphoreType.DMA,
            pltpu.SemaphoreType.DMA,
        ],
    )
    def sc_fn(xw_hbm, gidx_hbm, dst_hbm, zeros_hbm, out_hbm,
              idx_all, dst_all, rows, acc, gsem, ssem):
        c = lax.axis_index("c")
        s = lax.axis_index("s")
        wid = c * _NS + s
        pltpu.sync_copy(gidx_hbm.at[wid], idx_all)
        pltpu.sync_copy(dst_hbm.at[wid], dst_all)
        pltpu.sync_copy(zeros_hbm, acc.at[pl.ds(s * _RPT, _RPT)])
        plsc.subcore_barrier()

        def g_start(g, b):
            pltpu.async_copy(xw_hbm.at[idx_all.at[g]], rows.at[b], gsem)

        def g_wait(b):
            pltpu.make_async_copy(xw_hbm.at[idx_all.at[0]], rows.at[b],
                                  gsem).wait()

        def s_start(g, b):
            pltpu.async_copy(rows.at[b], acc.at[dst_all.at[g]], ssem,
                             add=True)

        def s_wait(b):
            pltpu.make_async_copy(rows.at[b], acc.at[dst_all.at[0]],
                                  ssem).wait()

        def substep(g, j, do_swait, do_gstart):
            nxt = g + _NBUF - 1
            nb = (j + _NBUF - 1) % _NBUF
            if do_gstart:
                if do_swait:
                    s_wait(nb)
                g_start(nxt, nb)
            g_wait(j)
            s_start(g, j)

        for b in range(_NBUF - 1):
            g_start(b, b)
        for g in range(_NBUF):
            substep(g, g, do_swait=(g >= 1), do_gstart=True)

        def body(i, carry):
            for j in range(_NBUF):
                substep(i * _NBUF + j, j, do_swait=True, do_gstart=True)
            return carry

        lax.fori_loop(1, _NCHUNK // _NBUF - 1, body, 0)
        base = _NCHUNK - _NBUF
        for j in range(_NBUF):
            g = base + j
            in_range = g + _NBUF - 1 < _NCHUNK
            substep(g, j, do_swait=in_range, do_gstart=in_range)
        for j in range(_NBUF):
            s_wait(j)

        plsc.subcore_barrier()
        pltpu.sync_copy(acc.at[pl.ds(s * _RPT, _RPT)],
                        out_hbm.at[c, pl.ds(s * _RPT, _RPT)])

    return sc_fn


def _sconv_sc(xw, gidx, dst, zeros):
    O = xw.shape[-1]
    fn = _make_sc_scatter(O)
    return fn(xw.reshape(K * N, O),
              gidx.reshape(_NW, _NCHUNK, _B),
              dst.reshape(_NW, _NCHUNK, _B), zeros)


def kernel(x, edge_index, edge_kpos, W_conv1, g0, b0, W1a, g1a, b1a, W1b,
           g1b, b1b, W2a, g2a, b2a, W2b, g2b, b2b, W2d, g2d, b2d, W3a, g3a,
           b3a, W3b, g3b, b3b, W3d, g3d, b3d, W_conv2):
    gidx, dst = _prep_indices(edge_index, edge_kpos)
    z32 = jnp.zeros((_RPT, 32), jnp.float32)
    z64 = jnp.zeros((_RPT, 64), jnp.float32)
    z96 = jnp.zeros((_RPT, 96), jnp.float32)

    def sconv(h, W, zeros):
        return _sconv_sc(_mm_all_k(h, W), gidx, dst, zeros)

    u0 = sconv(x, W_conv1, z32)
    h0 = _act(u0, g0, b0)

    u1a = sconv(h0, W1a, z32)
    h1a = _act(u1a, g1a, b1a)
    u1b = sconv(h1a, W1b, z32)
    h1, r2 = _act(u1b, g1b, b1b, res=h0, Wd=W2d)

    u2a = sconv(h1, W2a, z64)
    h2a = _act(u2a, g2a, b2a)
    u2b = sconv(h2a, W2b, z64)
    h2, r3 = _act(u2b, g2b, b2b, rbn=(r2, g2d, b2d), Wd=W3d)

    u3a = sconv(h2, W3a, z96)
    h3a = _act(u3a, g3a, b3a)
    u3b = sconv(h3a, W3b, z96)

    return _final(u3b, g3b, b3b, r3, g3d, b3d, W_conv2)
